# baseline (speedup 1.0000x reference)
import functools

import jax
import jax.numpy as jnp
from jax import lax
from jax.experimental import pallas as pl
from jax.experimental.pallas import tpu as pltpu
from jax.experimental.pallas import tpu_sc as plsc

_MAXREL = 32
_SEQ = 512
_DIM = 128
_NTAB = 2 * _MAXREL + 1
_BAND = 136
_PAD = (_BAND - 72) // 2 + _MAXREL
_ROWS_PER_TILE = 16
_CROWS_PER_TILE = 32
_INROWS = _BAND + 24


def _zone_geom(i):
    j0 = jnp.clip(i - _PAD, 0, _SEQ - _BAND)
    j0a = (j0 // 8) * 8
    l_hi = j0a
    l_lo = _SEQ - _BAND - j0a
    a_hi = (l_hi // 64) * 64
    a_lo = (l_lo // 64) * 64
    return l_hi, l_lo, a_hi, a_lo, j0a


def _interior_chunks(i, a_hi, a_lo):
    chunks = []
    n_hi = a_hi // 64
    b4 = (n_hi // 4) % 2
    b2 = (n_hi // 2) % 2
    b1 = n_hi % 2
    off2 = b4 * 256
    off1 = off2 + b2 * 128
    chunks.append(("hi", b4 != 0, jnp.int32(0), 256))
    chunks.append(("hi", b2 != 0, off2, 128))
    chunks.append(("hi", b1 != 0, off1, 64))
    n_lo = a_lo // 64
    c4 = (n_lo // 4) % 2
    c2 = (n_lo // 2) % 2
    c1 = n_lo % 2
    p4 = _SEQ - c4 * 256
    p2 = p4 - c2 * 128
    p1 = p2 - c1 * 64
    chunks.append(("lo", c4 != 0, jnp.minimum(p4, _SEQ - 256), 256))
    chunks.append(("lo", c2 != 0, jnp.minimum(p2, _SEQ - 128), 128))
    chunks.append(("lo", c1 != 0, jnp.minimum(p1, _SEQ - 64), 64))
    return chunks


def _sc_body(in_hbm, tab_hbm, out_hbm,
             in_v, pin_v, tab_v, cbuf_hi, cbuf_lo, band_v, chi_s, clo_s,
             sem_bulk, sem_edge, sem_in, sem_b0, sem_b1):
    c = lax.axis_index("c")
    s = lax.axis_index("s")
    wid = c * 16 + s
    row0 = wid * _ROWS_PER_TILE
    base_b = jnp.clip(row0 - _PAD - 8, 0, _SEQ - _INROWS)
    base_b = pl.multiple_of(base_b, 8)

    in_cp = pltpu.make_async_copy(in_hbm.at[pl.ds(base_b, _INROWS)], in_v,
                                  sem_in)
    in_cp.start()
    pltpu.sync_copy(tab_hbm, tab_v)
    pltpu.sync_copy(in_hbm.at[pl.ds(s * _CROWS_PER_TILE, _CROWS_PER_TILE)],
                    pin_v)

    def c_row(j, _):
        for cc in range(_DIM // 16):
            dsl = pl.ds(cc * 16, 16)
            x = pin_v[j, dsl]
            cbuf_hi[j, dsl] = x + tab_v[_NTAB - 1, dsl]
            cbuf_lo[j, dsl] = x + tab_v[0, dsl]
        return 0

    lax.fori_loop(0, _CROWS_PER_TILE, c_row, 0)
    pltpu.sync_copy(cbuf_hi, chi_s.at[pl.ds(s * _CROWS_PER_TILE, _CROWS_PER_TILE)])
    pltpu.sync_copy(cbuf_lo, clo_s.at[pl.ds(s * _CROWS_PER_TILE, _CROWS_PER_TILE)])
    plsc.subcore_barrier()
    in_cp.wait()

    def fire_interior(i, a_hi, a_lo, do_wait):
        for side, cond, off, size in _interior_chunks(i, a_hi, a_lo):
            src = chi_s if side == "hi" else clo_s

            @pl.when(cond)
            def _(off=off, size=size, src=src):
                cp = pltpu.make_async_copy(
                    src.at[pl.ds(off, size)],
                    out_hbm.at[i, pl.ds(off, size)], sem_bulk)
                if do_wait:
                    cp.wait()
                else:
                    cp.start()

    def band_dma(r, do_wait):
        i = row0 + r
        _, _, _, _, j0a = _zone_geom(i)
        p = lax.rem(r, 2)

        @pl.when(p == 0)
        def _():
            cp = pltpu.make_async_copy(
                band_v.at[0], out_hbm.at[i, pl.ds(j0a, _BAND)], sem_b0)
            if do_wait:
                cp.wait()
            else:
                cp.start()

        @pl.when(p == 1)
        def _():
            cp = pltpu.make_async_copy(
                band_v.at[1], out_hbm.at[i, pl.ds(j0a, _BAND)], sem_b1)
            if do_wait:
                cp.wait()
            else:
                cp.start()

    def out_row(r, _):
        i = row0 + r
        l_hi, l_lo, a_hi, a_lo, j0a = _zone_geom(i)
        p = lax.rem(r, 2)

        fire_interior(i, a_hi, a_lo, do_wait=False)

        eh = l_hi > a_hi
        el = l_lo > a_lo
        po = _SEQ - a_lo - 64

        @pl.when(eh)
        def _():
            pltpu.async_copy(chi_s.at[pl.ds(a_hi, 64)],
                             out_hbm.at[i, pl.ds(a_hi, 64)], sem_edge)

        @pl.when(el)
        def _():
            pltpu.async_copy(clo_s.at[pl.ds(po, 64)],
                             out_hbm.at[i, pl.ds(po, 64)], sem_edge)

        @pl.when(r >= 2)
        def _():
            band_dma(r - 2, do_wait=True)

        def band_row(t, _):
            j = j0a + t
            ridx = jnp.clip(i - j, -_MAXREL, _MAXREL) + _MAXREL
            for cc in range(_DIM // 16):
                dsl = pl.ds(cc * 16, 16)
                band_v[p, t, dsl] = in_v[j - base_b, dsl] + tab_v[ridx, dsl]
            return 0

        lax.fori_loop(0, _BAND, band_row, 0)

        @pl.when(eh)
        def _():
            pltpu.make_async_copy(chi_s.at[pl.ds(a_hi, 64)],
                                  out_hbm.at[i, pl.ds(a_hi, 64)],
                                  sem_edge).wait()

        @pl.when(el)
        def _():
            pltpu.make_async_copy(clo_s.at[pl.ds(po, 64)],
                                  out_hbm.at[i, pl.ds(po, 64)],
                                  sem_edge).wait()

        band_dma(r, do_wait=False)
        return 0

    lax.fori_loop(0, _ROWS_PER_TILE, out_row, 0)

    band_dma(_ROWS_PER_TILE - 2, do_wait=True)
    band_dma(_ROWS_PER_TILE - 1, do_wait=True)

    def drain_row(r, _):
        i = row0 + r
        _, _, a_hi, a_lo, _ = _zone_geom(i)
        fire_interior(i, a_hi, a_lo, do_wait=True)
        return 0

    lax.fori_loop(0, _ROWS_PER_TILE, drain_row, 0)


@jax.jit
def _rpe_sc(in2d, table):
    mesh = plsc.VectorSubcoreMesh(core_axis_name="c", subcore_axis_name="s")
    f = functools.partial(
        pl.kernel,
        out_type=jax.ShapeDtypeStruct((_SEQ, _SEQ, _DIM), jnp.float32),
        mesh=mesh,
        scratch_types=[
            pltpu.VMEM((_INROWS, _DIM), jnp.float32),
            pltpu.VMEM((_CROWS_PER_TILE, _DIM), jnp.float32),
            pltpu.VMEM((_NTAB, _DIM), jnp.float32),
            pltpu.VMEM((_CROWS_PER_TILE, _DIM), jnp.float32),
            pltpu.VMEM((_CROWS_PER_TILE, _DIM), jnp.float32),
            pltpu.VMEM((2, _BAND, _DIM), jnp.float32),
            pltpu.VMEM_SHARED((_SEQ, _DIM), jnp.float32),
            pltpu.VMEM_SHARED((_SEQ, _DIM), jnp.float32),
            pltpu.SemaphoreType.DMA,
            pltpu.SemaphoreType.DMA,
            pltpu.SemaphoreType.DMA,
            pltpu.SemaphoreType.DMA,
            pltpu.SemaphoreType.DMA,
        ],
    )(_sc_body)
    return f(in2d, table)


def kernel(input_embeddings, relative_position_embeddings):
    in2d = input_embeddings.reshape(_SEQ, _DIM)
    return _rpe_sc(in2d, relative_position_embeddings)


# speedup vs baseline: 1.2764x; 1.2764x over previous
import functools

import jax
import jax.numpy as jnp
from jax import lax
from jax.experimental import pallas as pl
from jax.experimental.pallas import tpu as pltpu
from jax.experimental.pallas import tpu_sc as plsc

_MAXREL = 32
_SEQ = 512
_DIM = 128
_NTAB = 2 * _MAXREL + 1
_BAND = 72
_PAD = (_BAND - 72) // 2 + _MAXREL
_ROWS_PER_TILE = 16
_CROWS_PER_TILE = 32
_INROWS = _BAND + 24


def _zone_geom(i):
    j0 = jnp.clip(i - _PAD, 0, _SEQ - _BAND)
    j0a = (j0 // 8) * 8
    l_hi = j0a
    l_lo = _SEQ - _BAND - j0a
    a_hi = (l_hi // 64) * 64
    a_lo = (l_lo // 64) * 64
    return l_hi, l_lo, a_hi, a_lo, j0a


def _interior_chunks(i, a_hi, a_lo):
    chunks = []
    n_hi = a_hi // 64
    b4 = (n_hi // 4) % 2
    b2 = (n_hi // 2) % 2
    b1 = n_hi % 2
    off2 = b4 * 256
    off1 = off2 + b2 * 128
    chunks.append(("hi", b4 != 0, jnp.int32(0), 256))
    chunks.append(("hi", b2 != 0, off2, 128))
    chunks.append(("hi", b1 != 0, off1, 64))
    n_lo = a_lo // 64
    c4 = (n_lo // 4) % 2
    c2 = (n_lo // 2) % 2
    c1 = n_lo % 2
    p4 = _SEQ - c4 * 256
    p2 = p4 - c2 * 128
    p1 = p2 - c1 * 64
    chunks.append(("lo", c4 != 0, jnp.minimum(p4, _SEQ - 256), 256))
    chunks.append(("lo", c2 != 0, jnp.minimum(p2, _SEQ - 128), 128))
    chunks.append(("lo", c1 != 0, jnp.minimum(p1, _SEQ - 64), 64))
    return chunks


def _sc_body(in_hbm, tab_hbm, out_hbm,
             in_v, pin_v, tab_v, cbuf_hi, cbuf_lo, band_v, chi_s, clo_s,
             sem_bulk, sem_edge, sem_in, sem_b0, sem_b1):
    c = lax.axis_index("c")
    s = lax.axis_index("s")
    wid = c * 16 + s
    row0 = wid * _ROWS_PER_TILE
    base_b = jnp.clip(row0 - _PAD - 8, 0, _SEQ - _INROWS)
    base_b = pl.multiple_of(base_b, 8)

    in_cp = pltpu.make_async_copy(in_hbm.at[pl.ds(base_b, _INROWS)], in_v,
                                  sem_in)
    in_cp.start()
    pltpu.sync_copy(tab_hbm, tab_v)
    pltpu.sync_copy(in_hbm.at[pl.ds(s * _CROWS_PER_TILE, _CROWS_PER_TILE)],
                    pin_v)

    def c_row(j, _):
        for cc in range(_DIM // 16):
            dsl = pl.ds(cc * 16, 16)
            x = pin_v[j, dsl]
            cbuf_hi[j, dsl] = x + tab_v[_NTAB - 1, dsl]
            cbuf_lo[j, dsl] = x + tab_v[0, dsl]
        return 0

    lax.fori_loop(0, _CROWS_PER_TILE, c_row, 0)
    pltpu.sync_copy(cbuf_hi, chi_s.at[pl.ds(s * _CROWS_PER_TILE, _CROWS_PER_TILE)])
    pltpu.sync_copy(cbuf_lo, clo_s.at[pl.ds(s * _CROWS_PER_TILE, _CROWS_PER_TILE)])
    plsc.subcore_barrier()
    in_cp.wait()

    def fire_interior(i, a_hi, a_lo, do_wait):
        for side, cond, off, size in _interior_chunks(i, a_hi, a_lo):
            src = chi_s if side == "hi" else clo_s

            @pl.when(cond)
            def _(off=off, size=size, src=src):
                cp = pltpu.make_async_copy(
                    src.at[pl.ds(off, size)],
                    out_hbm.at[i, pl.ds(off, size)], sem_bulk)
                if do_wait:
                    cp.wait()
                else:
                    cp.start()

    def band_dma(r, do_wait):
        i = row0 + r
        _, _, _, _, j0a = _zone_geom(i)
        p = lax.rem(r, 2)

        @pl.when(p == 0)
        def _():
            cp = pltpu.make_async_copy(
                band_v.at[0], out_hbm.at[i, pl.ds(j0a, _BAND)], sem_b0)
            if do_wait:
                cp.wait()
            else:
                cp.start()

        @pl.when(p == 1)
        def _():
            cp = pltpu.make_async_copy(
                band_v.at[1], out_hbm.at[i, pl.ds(j0a, _BAND)], sem_b1)
            if do_wait:
                cp.wait()
            else:
                cp.start()

    def out_row(r, _):
        i = row0 + r
        l_hi, l_lo, a_hi, a_lo, j0a = _zone_geom(i)
        p = lax.rem(r, 2)

        fire_interior(i, a_hi, a_lo, do_wait=False)

        eh = l_hi > a_hi
        el = l_lo > a_lo
        po = _SEQ - a_lo - 64

        @pl.when(eh)
        def _():
            pltpu.async_copy(chi_s.at[pl.ds(a_hi, 64)],
                             out_hbm.at[i, pl.ds(a_hi, 64)], sem_edge)

        @pl.when(el)
        def _():
            pltpu.async_copy(clo_s.at[pl.ds(po, 64)],
                             out_hbm.at[i, pl.ds(po, 64)], sem_edge)

        @pl.when(r >= 2)
        def _():
            band_dma(r - 2, do_wait=True)

        def band_row(t, _):
            j = j0a + t
            ridx = jnp.clip(i - j, -_MAXREL, _MAXREL) + _MAXREL
            for cc in range(_DIM // 16):
                dsl = pl.ds(cc * 16, 16)
                band_v[p, t, dsl] = in_v[j - base_b, dsl] + tab_v[ridx, dsl]
            return 0

        lax.fori_loop(0, _BAND, band_row, 0)

        @pl.when(eh)
        def _():
            pltpu.make_async_copy(chi_s.at[pl.ds(a_hi, 64)],
                                  out_hbm.at[i, pl.ds(a_hi, 64)],
                                  sem_edge).wait()

        @pl.when(el)
        def _():
            pltpu.make_async_copy(clo_s.at[pl.ds(po, 64)],
                                  out_hbm.at[i, pl.ds(po, 64)],
                                  sem_edge).wait()

        band_dma(r, do_wait=False)
        return 0

    lax.fori_loop(0, _ROWS_PER_TILE, out_row, 0)

    band_dma(_ROWS_PER_TILE - 2, do_wait=True)
    band_dma(_ROWS_PER_TILE - 1, do_wait=True)

    def drain_row(r, _):
        i = row0 + r
        _, _, a_hi, a_lo, _ = _zone_geom(i)
        fire_interior(i, a_hi, a_lo, do_wait=True)
        return 0

    lax.fori_loop(0, _ROWS_PER_TILE, drain_row, 0)


@jax.jit
def _rpe_sc(in2d, table):
    mesh = plsc.VectorSubcoreMesh(core_axis_name="c", subcore_axis_name="s")
    f = functools.partial(
        pl.kernel,
        out_type=jax.ShapeDtypeStruct((_SEQ, _SEQ, _DIM), jnp.float32),
        mesh=mesh,
        scratch_types=[
            pltpu.VMEM((_INROWS, _DIM), jnp.float32),
            pltpu.VMEM((_CROWS_PER_TILE, _DIM), jnp.float32),
            pltpu.VMEM((_NTAB, _DIM), jnp.float32),
            pltpu.VMEM((_CROWS_PER_TILE, _DIM), jnp.float32),
            pltpu.VMEM((_CROWS_PER_TILE, _DIM), jnp.float32),
            pltpu.VMEM((2, _BAND, _DIM), jnp.float32),
            pltpu.VMEM_SHARED((_SEQ, _DIM), jnp.float32),
            pltpu.VMEM_SHARED((_SEQ, _DIM), jnp.float32),
            pltpu.SemaphoreType.DMA,
            pltpu.SemaphoreType.DMA,
            pltpu.SemaphoreType.DMA,
            pltpu.SemaphoreType.DMA,
            pltpu.SemaphoreType.DMA,
        ],
    )(_sc_body)
    return f(in2d, table)


def kernel(input_embeddings, relative_position_embeddings):
    in2d = input_embeddings.reshape(_SEQ, _DIM)
    return _rpe_sc(in2d, relative_position_embeddings)


# speedup vs baseline: 1.3872x; 1.0868x over previous
import functools

import jax
import jax.numpy as jnp
from jax import lax
from jax.experimental import pallas as pl
from jax.experimental.pallas import tpu as pltpu
from jax.experimental.pallas import tpu_sc as plsc

_MAXREL = 32
_SEQ = 512
_DIM = 128
_NTAB = 2 * _MAXREL + 1
_BAND = 72
_PAD = (_BAND - 72) // 2 + _MAXREL
_ROWS_PER_TILE = 16
_CROWS_PER_TILE = 32
_INROWS = _BAND + 24


def _zone_geom(i):
    j0 = jnp.clip(i - _PAD, 0, _SEQ - _BAND)
    j0a = (j0 // 8) * 8
    l_hi = j0a
    l_lo = _SEQ - _BAND - j0a
    return l_hi, l_lo, j0a


def _bits(n):
    return (n // 4) % 2, (n // 2) % 2, n % 2


def _interior_chunks(l_hi, l_lo):
    chunks = []
    off = jnp.int32(0)
    rem = l_hi
    for size in (256, 128, 64, 32, 16, 8):
        bit = (rem // size) % 2
        chunks.append(("hi", bit != 0, off, size))
        off = off + bit * size
        rem = rem
    off = jnp.int32(_SEQ)
    rem = l_lo
    for size in (256, 128, 64, 32, 16, 8):
        bit = (rem // size) % 2
        off = off - bit * size
        chunks.append(("lo", bit != 0, jnp.minimum(off, _SEQ - size), size))
    return chunks


def _sc_body(in_hbm, tab_hbm, out_hbm,
             in_v, pin_v, tab_v, cbuf_hi, cbuf_lo, band_v, chi_s, clo_s,
             sem_bulk, sem_in, sem_b0, sem_b1):
    c = lax.axis_index("c")
    s = lax.axis_index("s")
    wid = c * 16 + s
    row0 = wid * _ROWS_PER_TILE
    base_b = jnp.clip(row0 - _PAD - 8, 0, _SEQ - _INROWS)
    base_b = pl.multiple_of(base_b, 8)

    in_cp = pltpu.make_async_copy(in_hbm.at[pl.ds(base_b, _INROWS)], in_v,
                                  sem_in)
    in_cp.start()
    pltpu.sync_copy(tab_hbm, tab_v)
    pltpu.sync_copy(in_hbm.at[pl.ds(s * _CROWS_PER_TILE, _CROWS_PER_TILE)],
                    pin_v)

    def c_row(j, _):
        for cc in range(_DIM // 16):
            dsl = pl.ds(cc * 16, 16)
            x = pin_v[j, dsl]
            cbuf_hi[j, dsl] = x + tab_v[_NTAB - 1, dsl]
            cbuf_lo[j, dsl] = x + tab_v[0, dsl]
        return 0

    lax.fori_loop(0, _CROWS_PER_TILE, c_row, 0)
    pltpu.sync_copy(cbuf_hi, chi_s.at[pl.ds(s * _CROWS_PER_TILE, _CROWS_PER_TILE)])
    pltpu.sync_copy(cbuf_lo, clo_s.at[pl.ds(s * _CROWS_PER_TILE, _CROWS_PER_TILE)])
    plsc.subcore_barrier()
    in_cp.wait()

    def fire_interior(i, l_hi, l_lo, do_wait):
        for side, cond, off, size in _interior_chunks(l_hi, l_lo):
            src = chi_s if side == "hi" else clo_s
            off = pl.multiple_of(off, 8)

            @pl.when(cond)
            def _(off=off, size=size, src=src):
                cp = pltpu.make_async_copy(
                    src.at[pl.ds(off, size)],
                    out_hbm.at[i, pl.ds(off, size)], sem_bulk)
                if do_wait:
                    cp.wait()
                else:
                    cp.start()

    def band_dma(r, do_wait):
        i = row0 + r
        _, _, j0a = _zone_geom(i)
        p = lax.rem(r, 2)

        @pl.when(p == 0)
        def _():
            cp = pltpu.make_async_copy(
                band_v.at[0], out_hbm.at[i, pl.ds(j0a, _BAND)], sem_b0)
            if do_wait:
                cp.wait()
            else:
                cp.start()

        @pl.when(p == 1)
        def _():
            cp = pltpu.make_async_copy(
                band_v.at[1], out_hbm.at[i, pl.ds(j0a, _BAND)], sem_b1)
            if do_wait:
                cp.wait()
            else:
                cp.start()

    def out_row(r, _):
        i = row0 + r
        l_hi, l_lo, j0a = _zone_geom(i)
        p = lax.rem(r, 2)

        fire_interior(i, l_hi, l_lo, do_wait=False)

        @pl.when(r >= 2)
        def _():
            band_dma(r - 2, do_wait=True)

        def band_row(t, _):
            j = j0a + t
            ridx = jnp.clip(i - j, -_MAXREL, _MAXREL) + _MAXREL
            for cc in range(_DIM // 16):
                dsl = pl.ds(cc * 16, 16)
                band_v[p, t, dsl] = in_v[j - base_b, dsl] + tab_v[ridx, dsl]
            return 0

        lax.fori_loop(0, _BAND, band_row, 0)
        band_dma(r, do_wait=False)
        return 0

    lax.fori_loop(0, _ROWS_PER_TILE, out_row, 0)

    band_dma(_ROWS_PER_TILE - 2, do_wait=True)
    band_dma(_ROWS_PER_TILE - 1, do_wait=True)

    def drain_row(r, _):
        i = row0 + r
        l_hi, l_lo, _ = _zone_geom(i)
        fire_interior(i, l_hi, l_lo, do_wait=True)
        return 0

    lax.fori_loop(0, _ROWS_PER_TILE, drain_row, 0)


@jax.jit
def _rpe_sc(in2d, table):
    mesh = plsc.VectorSubcoreMesh(core_axis_name="c", subcore_axis_name="s")
    f = functools.partial(
        pl.kernel,
        out_type=jax.ShapeDtypeStruct((_SEQ, _SEQ, _DIM), jnp.float32),
        mesh=mesh,
        scratch_types=[
            pltpu.VMEM((_INROWS, _DIM), jnp.float32),
            pltpu.VMEM((_CROWS_PER_TILE, _DIM), jnp.float32),
            pltpu.VMEM((_NTAB, _DIM), jnp.float32),
            pltpu.VMEM((_CROWS_PER_TILE, _DIM), jnp.float32),
            pltpu.VMEM((_CROWS_PER_TILE, _DIM), jnp.float32),
            pltpu.VMEM((2, _BAND, _DIM), jnp.float32),
            pltpu.VMEM_SHARED((_SEQ, _DIM), jnp.float32),
            pltpu.VMEM_SHARED((_SEQ, _DIM), jnp.float32),
            pltpu.SemaphoreType.DMA,
            pltpu.SemaphoreType.DMA,
            pltpu.SemaphoreType.DMA,
            pltpu.SemaphoreType.DMA,
        ],
    )(_sc_body)
    return f(in2d, table)


def kernel(input_embeddings, relative_position_embeddings):
    in2d = input_embeddings.reshape(_SEQ, _DIM)
    return _rpe_sc(in2d, relative_position_embeddings)


# speedup vs baseline: 1.3879x; 1.0005x over previous
import functools

import jax
import jax.numpy as jnp
from jax import lax
from jax.experimental import pallas as pl
from jax.experimental.pallas import tpu as pltpu
from jax.experimental.pallas import tpu_sc as plsc

_MAXREL = 32
_SEQ = 512
_DIM = 128
_NTAB = 2 * _MAXREL + 1
_BAND = 72
_PAD = (_BAND - 72) // 2 + _MAXREL
_ROWS_PER_TILE = 16
_CROWS_PER_TILE = 32
_INROWS = _BAND + 24


def _zone_geom(i):
    j0 = jnp.clip(i - _PAD, 0, _SEQ - _BAND)
    j0a = (j0 // 8) * 8
    l_hi = j0a
    l_lo = _SEQ - _BAND - j0a
    return l_hi, l_lo, j0a


def _bits(n):
    return (n // 4) % 2, (n // 2) % 2, n % 2


def _interior_chunks(l_hi, l_lo):
    chunks = []
    off = jnp.int32(0)
    for size in (256, 128, 64, 32, 16, 8):
        bit = (l_hi // size) % 2
        chunks.append(("hi", bit != 0, off, size))
        off = off + bit * size
    off = jnp.int32(_SEQ)
    for size in (256, 128, 64, 32, 16, 8):
        bit = (l_lo // size) % 2
        off = off - bit * size
        chunks.append(("lo", bit != 0, jnp.minimum(off, _SEQ - size), size))
    return chunks


def _sc_body(in_hbm, tab_hbm, out_hbm,
             in_v, pin_v, tab_v, cbuf_hi, cbuf_lo, band_v, chi_s, clo_s,
             sem_bulk, sem_in, sem_b0, sem_b1):
    c = lax.axis_index("c")
    s = lax.axis_index("s")
    wid = c * 16 + s
    row0 = wid * _ROWS_PER_TILE
    base_b = jnp.clip(row0 - _PAD - 8, 0, _SEQ - _INROWS)
    base_b = pl.multiple_of(base_b, 8)

    in_cp = pltpu.make_async_copy(in_hbm.at[pl.ds(base_b, _INROWS)], in_v,
                                  sem_in)
    in_cp.start()
    pltpu.sync_copy(tab_hbm, tab_v)
    pltpu.sync_copy(in_hbm.at[pl.ds(s * _CROWS_PER_TILE, _CROWS_PER_TILE)],
                    pin_v)

    def c_row(j, _):
        for cc in range(_DIM // 16):
            dsl = pl.ds(cc * 16, 16)
            x = pin_v[j, dsl]
            cbuf_hi[j, dsl] = x + tab_v[_NTAB - 1, dsl]
            cbuf_lo[j, dsl] = x + tab_v[0, dsl]
        return 0

    lax.fori_loop(0, _CROWS_PER_TILE, c_row, 0)
    pltpu.sync_copy(cbuf_hi, chi_s.at[pl.ds(s * _CROWS_PER_TILE, _CROWS_PER_TILE)])
    pltpu.sync_copy(cbuf_lo, clo_s.at[pl.ds(s * _CROWS_PER_TILE, _CROWS_PER_TILE)])
    plsc.subcore_barrier()
    in_cp.wait()

    def fire_interior(i, l_hi, l_lo, do_wait):
        for side, cond, off, size in _interior_chunks(l_hi, l_lo):
            src = chi_s if side == "hi" else clo_s
            off = pl.multiple_of(off, 8)

            @pl.when(cond)
            def _(off=off, size=size, src=src):
                cp = pltpu.make_async_copy(
                    src.at[pl.ds(off, size)],
                    out_hbm.at[i, pl.ds(off, size)], sem_bulk)
                if do_wait:
                    cp.wait()
                else:
                    cp.start()

    def band_dma(r, do_wait):
        i = row0 + r
        _, _, j0a = _zone_geom(i)
        p = lax.rem(r, 2)

        @pl.when(p == 0)
        def _():
            cp = pltpu.make_async_copy(
                band_v.at[0], out_hbm.at[i, pl.ds(j0a, _BAND)], sem_b0)
            if do_wait:
                cp.wait()
            else:
                cp.start()

        @pl.when(p == 1)
        def _():
            cp = pltpu.make_async_copy(
                band_v.at[1], out_hbm.at[i, pl.ds(j0a, _BAND)], sem_b1)
            if do_wait:
                cp.wait()
            else:
                cp.start()

    def out_row(r, _):
        i = row0 + r
        l_hi, l_lo, j0a = _zone_geom(i)
        p = lax.rem(r, 2)

        fire_interior(i, l_hi, l_lo, do_wait=False)

        @pl.when(r >= 2)
        def _():
            band_dma(r - 2, do_wait=True)

        def band_row(t, _):
            j = j0a + t
            ridx = jnp.clip(i - j, -_MAXREL, _MAXREL) + _MAXREL
            for cc in range(_DIM // 16):
                dsl = pl.ds(cc * 16, 16)
                band_v[p, t, dsl] = in_v[j - base_b, dsl] + tab_v[ridx, dsl]
            return 0

        lax.fori_loop(0, _BAND, band_row, 0)
        band_dma(r, do_wait=False)
        return 0

    lax.fori_loop(0, _ROWS_PER_TILE, out_row, 0)

    band_dma(_ROWS_PER_TILE - 2, do_wait=True)
    band_dma(_ROWS_PER_TILE - 1, do_wait=True)

    def drain_row(r, _):
        i = row0 + r
        l_hi, l_lo, _ = _zone_geom(i)
        fire_interior(i, l_hi, l_lo, do_wait=True)
        return 0

    lax.fori_loop(0, _ROWS_PER_TILE, drain_row, 0)


@jax.jit
def _rpe_sc(in2d, table):
    mesh = plsc.VectorSubcoreMesh(core_axis_name="c", subcore_axis_name="s")
    f = functools.partial(
        pl.kernel,
        out_type=jax.ShapeDtypeStruct((_SEQ, _SEQ, _DIM), jnp.float32),
        mesh=mesh,
        scratch_types=[
            pltpu.VMEM((_INROWS, _DIM), jnp.float32),
            pltpu.VMEM((_CROWS_PER_TILE, _DIM), jnp.float32),
            pltpu.VMEM((_NTAB, _DIM), jnp.float32),
            pltpu.VMEM((_CROWS_PER_TILE, _DIM), jnp.float32),
            pltpu.VMEM((_CROWS_PER_TILE, _DIM), jnp.float32),
            pltpu.VMEM((2, _BAND, _DIM), jnp.float32),
            pltpu.VMEM_SHARED((_SEQ, _DIM), jnp.float32),
            pltpu.VMEM_SHARED((_SEQ, _DIM), jnp.float32),
            pltpu.SemaphoreType.DMA,
            pltpu.SemaphoreType.DMA,
            pltpu.SemaphoreType.DMA,
            pltpu.SemaphoreType.DMA,
        ],
    )(_sc_body)
    return f(in2d, table)


def kernel(input_embeddings, relative_position_embeddings):
    in2d = input_embeddings.reshape(_SEQ, _DIM)
    return _rpe_sc(in2d, relative_position_embeddings)
